# baseline (speedup 1.0000x reference)
import jax
import jax.numpy as jnp
import numpy as np
from jax.experimental import pallas as pl
from jax.experimental.pallas import tpu as pltpu

_ZDIM = 512
_LAYERS = 8
_NUM_WS = 14
_WS_SPLIT = 2
_WS_HALF = _NUM_WS // _WS_SPLIT
_WGAIN = 0.01 / np.sqrt(512.0)
_BGAIN = 0.01


def _mlp_kernel(z_ref, w_ref, b_ref, o_ref, wh_ref, x_ref):
    i = pl.program_id(0)
    j = pl.program_id(1)

    @pl.when((i == 0) & (j == 0))
    def _():
        wh_ref[...] = (w_ref[...] * _WGAIN).astype(jnp.bfloat16)

    @pl.when(j == 0)
    def _():
        x = z_ref[...]
        x = x * jax.lax.rsqrt(jnp.mean(x * x, axis=1, keepdims=True) + 1e-8)
        dims = (((1,), (1,)), ((), ()))
        for k in range(_LAYERS):
            y = jax.lax.dot_general(x.astype(jnp.bfloat16), wh_ref[k], dims,
                                    preferred_element_type=jnp.float32)
            y = y + b_ref[k][None, :] * _BGAIN
            x = jnp.where(y >= 0, y, 0.01 * y)
        x_ref[...] = x

    o_ref[...] = jnp.broadcast_to(
        x_ref[...][None, :, :], (_WS_HALF,) + x_ref.shape)


def kernel(z, c, W, b):
    del c
    batch = z.shape[0]
    m = 1024
    out = pl.pallas_call(
        _mlp_kernel,
        grid=(batch // m, _WS_SPLIT),
        in_specs=[
            pl.BlockSpec((m, _ZDIM), lambda i, j: (i, 0)),
            pl.BlockSpec((_LAYERS, _ZDIM, _ZDIM), lambda i, j: (0, 0, 0)),
            pl.BlockSpec((_LAYERS, _ZDIM), lambda i, j: (0, 0)),
        ],
        out_specs=pl.BlockSpec((_WS_HALF, m, _ZDIM), lambda i, j: (j, i, 0)),
        out_shape=jax.ShapeDtypeStruct((_NUM_WS, batch, _ZDIM), jnp.float32),
        scratch_shapes=[
            pltpu.VMEM((_LAYERS, _ZDIM, _ZDIM), jnp.bfloat16),
            pltpu.VMEM((m, _ZDIM), jnp.float32),
        ],
    )(z, W, b)
    return jnp.transpose(out, (1, 0, 2))


# speedup vs baseline: 1.1718x; 1.1718x over previous
import jax
import jax.numpy as jnp
import numpy as np
from jax.experimental import pallas as pl
from jax.experimental.pallas import tpu as pltpu

_ZDIM = 512
_LAYERS = 8
_NUM_WS = 14
_WGAIN = 0.01 / np.sqrt(512.0)
_BGAIN = 0.01


def _mlp_kernel(z_ref, w_ref, b_ref, o_ref, wh_ref):
    @pl.when(pl.program_id(0) == 0)
    def _():
        wh_ref[...] = (w_ref[...] * _WGAIN).astype(jnp.bfloat16)

    x = z_ref[...]
    x = x * jax.lax.rsqrt(jnp.mean(x * x, axis=1, keepdims=True) + 1e-8)
    dims = (((1,), (1,)), ((), ()))
    for i in range(_LAYERS):
        y = jax.lax.dot_general(x.astype(jnp.bfloat16), wh_ref[i], dims,
                                preferred_element_type=jnp.float32)
        y = y + b_ref[i][None, :] * _BGAIN
        x = jnp.where(y >= 0, y, 0.01 * y)
    o_ref[...] = jnp.broadcast_to(x[None, :, :], (_NUM_WS, x.shape[0], _ZDIM))


def kernel(z, c, W, b):
    del c
    batch = z.shape[0]
    m = 512
    out = pl.pallas_call(
        _mlp_kernel,
        grid=(batch // m,),
        in_specs=[
            pl.BlockSpec((m, _ZDIM), lambda i: (i, 0)),
            pl.BlockSpec((_LAYERS, _ZDIM, _ZDIM), lambda i: (0, 0, 0)),
            pl.BlockSpec((_LAYERS, _ZDIM), lambda i: (0, 0)),
        ],
        out_specs=pl.BlockSpec((_NUM_WS, m, _ZDIM), lambda i: (0, i, 0)),
        out_shape=jax.ShapeDtypeStruct((_NUM_WS, batch, _ZDIM), jnp.float32),
        scratch_shapes=[
            pltpu.VMEM((_LAYERS, _ZDIM, _ZDIM), jnp.bfloat16),
        ],
    )(z, W, b)
    return jnp.transpose(out, (1, 0, 2))
